# Initial kernel scaffold; baseline (speedup 1.0000x reference)
import functools

import jax
import jax.numpy as jnp
from jax import lax
from jax.experimental import pallas as pl
from jax.experimental.pallas import tpu as pltpu
from jax.experimental.pallas import tpu_sc as plsc

LRELU_SLOPE = 0.2

_NW = 32


def _sc_gather_rows(table, idx, n_chunks, chunk):
    npad, dp = idx.shape[0], table.shape[1]
    per_w = n_chunks * chunk
    assert npad == _NW * per_w
    mesh = plsc.VectorSubcoreMesh(core_axis_name="c", subcore_axis_name="s")

    @functools.partial(
        pl.kernel, mesh=mesh,
        out_type=jax.ShapeDtypeStruct((npad, dp), jnp.float32),
        scratch_types=[
            pltpu.VMEM((chunk,), jnp.int32),
            pltpu.VMEM((chunk, dp), jnp.float32),
            pltpu.SemaphoreType.DMA,
        ],
    )
    def k(table_hbm, idx_hbm, out_hbm, idx_v, rows_v, sem):
        wid = lax.axis_index("s") * 2 + lax.axis_index("c")

        @pl.loop(0, n_chunks)
        def _(c):
            base = wid * per_w + c * chunk
            pltpu.sync_copy(idx_hbm.at[pl.ds(base, chunk)], idx_v)
            pltpu.async_copy(table_hbm.at[idx_v], rows_v, sem).wait()
            pltpu.sync_copy(rows_v, out_hbm.at[pl.ds(base, chunk)])

    return k(table, idx)


def _tc_mlp(x, w1, b1, w2, b2, bn=2048):
    npad, dp = x.shape
    h1 = w1.shape[1]
    c = w2.shape[1]

    def body(x_ref, w1_ref, b1_ref, w2_ref, b2_ref, o_ref):
        h = jnp.dot(x_ref[...], w1_ref[...],
                    preferred_element_type=jnp.float32) + b1_ref[...]
        h = jnp.maximum(h, 0.0)
        o_ref[...] = jnp.dot(h, w2_ref[...],
                             preferred_element_type=jnp.float32) + b2_ref[...]

    return pl.pallas_call(
        body,
        grid=(npad // bn,),
        in_specs=[
            pl.BlockSpec((bn, dp), lambda i: (i, 0)),
            pl.BlockSpec((dp, h1), lambda i: (0, 0)),
            pl.BlockSpec((1, h1), lambda i: (0, 0)),
            pl.BlockSpec((h1, c), lambda i: (0, 0)),
            pl.BlockSpec((1, c), lambda i: (0, 0)),
        ],
        out_specs=pl.BlockSpec((bn, c), lambda i: (i, 0)),
        out_shape=jax.ShapeDtypeStruct((npad, c), jnp.float32),
    )(x, w1, b1, w2, b2)


def kernel(node_ids, edge_index, graph_ids, emb, W1, b1, W2, b2, Wg,
           a_src, a_dst, w_gate, b_gate):
    N = node_ids.shape[0]
    D = emb.shape[1]
    L, NH, C, _ = Wg.shape
    G = 512

    DP = ((D + 15) // 16) * 16
    NP = ((N + 255) // 256) * 256
    embp = jnp.pad(emb, ((0, 0), (0, DP - D)))
    idsp = jnp.pad(node_ids, (0, NP - N)).astype(jnp.int32)

    x = _sc_gather_rows(embp, idsp, n_chunks=16, chunk=NP // _NW // 16)

    w1p = jnp.pad(W1, ((0, DP - D), (0, 0)))
    h = _tc_mlp(x, w1p, b1[None, :], W2, b2[None, :])[:N]

    src = edge_index[0]
    dst = edge_index[1]

    for l in range(L):
        heads = []
        for hd in range(NH):
            z = h @ Wg[l, hd]
            e = jnp.take(z @ a_src[l, hd], src) + jnp.take(z @ a_dst[l, hd], dst)
            e = jnp.where(e > 0, e, LRELU_SLOPE * e)
            ee = jnp.exp(e)
            denom = jax.ops.segment_sum(ee, dst, num_segments=N)
            att = ee / (jnp.take(denom, dst) + 1e-9)
            msg = att[:, None] * jnp.take(z, src, axis=0)
            heads.append(jax.ops.segment_sum(msg, dst, num_segments=N))
        h = jax.nn.elu(jnp.mean(jnp.stack(heads, axis=0), axis=0))

    gate = jax.nn.sigmoid(h @ w_gate + b_gate)
    out = jax.ops.segment_sum(gate * h, graph_ids, num_segments=G)
    return out


# speedup vs baseline: 1.3090x; 1.3090x over previous
import functools

import jax
import jax.numpy as jnp
from jax import lax
from jax.experimental import pallas as pl
from jax.experimental.pallas import tpu as pltpu
from jax.experimental.pallas import tpu_sc as plsc

LRELU_SLOPE = 0.2

_NW = 32


def _sc_gather_rows(table, idx, n_chunks, chunk):
    npad, dp = idx.shape[0], table.shape[1]
    per_w = n_chunks * chunk
    assert npad == _NW * per_w
    mesh = plsc.VectorSubcoreMesh(core_axis_name="c", subcore_axis_name="s")

    @functools.partial(
        pl.kernel, mesh=mesh,
        out_type=jax.ShapeDtypeStruct((npad, dp), jnp.float32),
        scratch_types=[
            pltpu.VMEM((chunk,), jnp.int32),
            pltpu.VMEM((chunk, dp), jnp.float32),
            pltpu.SemaphoreType.DMA,
        ],
        compiler_params=pltpu.CompilerParams(use_tc_tiling_on_sc=False),
    )
    def k(table_hbm, idx_hbm, out_hbm, idx_v, rows_v, sem):
        wid = lax.axis_index("s") * 2 + lax.axis_index("c")

        @pl.loop(0, n_chunks)
        def _(c):
            base = wid * per_w + c * chunk
            pltpu.sync_copy(idx_hbm.at[pl.ds(base, chunk)], idx_v)
            pltpu.async_copy(table_hbm.at[idx_v], rows_v, sem).wait()
            pltpu.sync_copy(rows_v, out_hbm.at[pl.ds(base, chunk)])

    return k(table, idx)


def _tc_mlp(x, w1, b1, w2, b2, bn=2048):
    npad, dp = x.shape
    h1 = w1.shape[1]
    c = w2.shape[1]

    def body(x_ref, w1_ref, b1_ref, w2_ref, b2_ref, o_ref):
        h = jnp.dot(x_ref[...], w1_ref[...],
                    preferred_element_type=jnp.float32) + b1_ref[...]
        h = jnp.maximum(h, 0.0)
        o_ref[...] = jnp.dot(h, w2_ref[...],
                             preferred_element_type=jnp.float32) + b2_ref[...]

    return pl.pallas_call(
        body,
        grid=(npad // bn,),
        in_specs=[
            pl.BlockSpec((bn, dp), lambda i: (i, 0)),
            pl.BlockSpec((dp, h1), lambda i: (0, 0)),
            pl.BlockSpec((1, h1), lambda i: (0, 0)),
            pl.BlockSpec((h1, c), lambda i: (0, 0)),
            pl.BlockSpec((1, c), lambda i: (0, 0)),
        ],
        out_specs=pl.BlockSpec((bn, c), lambda i: (i, 0)),
        out_shape=jax.ShapeDtypeStruct((npad, c), jnp.float32),
    )(x, w1, b1, w2, b2)


def kernel(node_ids, edge_index, graph_ids, emb, W1, b1, W2, b2, Wg,
           a_src, a_dst, w_gate, b_gate):
    N = node_ids.shape[0]
    D = emb.shape[1]
    L, NH, C, _ = Wg.shape
    G = 512

    DP = ((D + 15) // 16) * 16
    NP = ((N + 4095) // 4096) * 4096
    embp = jnp.pad(emb, ((0, 0), (0, DP - D)))
    idsp = jnp.pad(node_ids, (0, NP - N)).astype(jnp.int32)

    x = _sc_gather_rows(embp, idsp, n_chunks=16, chunk=NP // _NW // 16)

    w1p = jnp.pad(W1, ((0, DP - D), (0, 0)))
    h = _tc_mlp(x, w1p, b1[None, :], W2, b2[None, :])[:N]

    src = edge_index[0]
    dst = edge_index[1]

    for l in range(L):
        heads = []
        for hd in range(NH):
            z = h @ Wg[l, hd]
            e = jnp.take(z @ a_src[l, hd], src) + jnp.take(z @ a_dst[l, hd], dst)
            e = jnp.where(e > 0, e, LRELU_SLOPE * e)
            ee = jnp.exp(e)
            denom = jax.ops.segment_sum(ee, dst, num_segments=N)
            att = ee / (jnp.take(denom, dst) + 1e-9)
            msg = att[:, None] * jnp.take(z, src, axis=0)
            heads.append(jax.ops.segment_sum(msg, dst, num_segments=N))
        h = jax.nn.elu(jnp.mean(jnp.stack(heads, axis=0), axis=0))

    gate = jax.nn.sigmoid(h @ w_gate + b_gate)
    out = jax.ops.segment_sum(gate * h, graph_ids, num_segments=G)
    return out


# speedup vs baseline: 14.5360x; 11.1045x over previous
import functools

import jax
import jax.numpy as jnp
from jax import lax
from jax.experimental import pallas as pl
from jax.experimental.pallas import tpu as pltpu
from jax.experimental.pallas import tpu_sc as plsc

LRELU_SLOPE = 0.2

_NW = 32
_NB = 3200
_NP = _NW * _NB
_CH = 256
_G = 512

_sc_params = pltpu.CompilerParams(needs_layout_passes=False,
                                 use_tc_tiling_on_sc=False)


def _mesh():
    return plsc.VectorSubcoreMesh(core_axis_name="c", subcore_axis_name="s")


def _wid():
    return lax.axis_index("s") * 2 + lax.axis_index("c")


def _iota16():
    return lax.iota(jnp.int32, 16)


def _zero_ref(ref, n):
    @pl.loop(0, n, step=16)
    def _(i):
        ref[pl.ds(i, 16)] = jnp.zeros((16,), jnp.float32)


def _sc_gather_rows(table, idx, n_chunks, chunk):
    npad, dp = idx.shape[0], table.shape[1]
    per_w = n_chunks * chunk
    assert npad == _NW * per_w

    @functools.partial(
        pl.kernel, mesh=_mesh(),
        out_type=jax.ShapeDtypeStruct((npad, dp), jnp.float32),
        scratch_types=[
            pltpu.VMEM((chunk,), jnp.int32),
            pltpu.VMEM((chunk, dp), jnp.float32),
            pltpu.SemaphoreType.DMA,
        ],
        compiler_params=_sc_params,
    )
    def k(table_hbm, idx_hbm, out_hbm, idx_v, rows_v, sem):
        wid = _wid()

        @pl.loop(0, n_chunks)
        def _(c):
            base = wid * per_w + c * chunk
            pltpu.sync_copy(idx_hbm.at[pl.ds(base, chunk)], idx_v)
            pltpu.async_copy(table_hbm.at[idx_v], rows_v, sem).wait()
            pltpu.sync_copy(rows_v, out_hbm.at[pl.ds(base, chunk)])

    return k(table, idx)


def _sc_bucket(src, dst):
    E = src.shape[0]
    CH2 = 2000
    n_ch = E // CH2
    assert n_ch * CH2 == E
    EP = E + _NW * _CH

    @functools.partial(
        pl.kernel, mesh=_mesh(),
        out_type=(jax.ShapeDtypeStruct((EP,), jnp.int32),
                  jax.ShapeDtypeStruct((EP,), jnp.int32),
                  jax.ShapeDtypeStruct((_NW * 16,), jnp.int32),
                  jax.ShapeDtypeStruct((_NW * 16,), jnp.int32)),
        scratch_types=[
            pltpu.VMEM((CH2,), jnp.int32),
            pltpu.VMEM((CH2,), jnp.int32),
            pltpu.VMEM((48,), jnp.int32),
            pltpu.VMEM((48,), jnp.int32),
            pltpu.VMEM((272,), jnp.int32),
            pltpu.VMEM((272,), jnp.int32),
            pltpu.VMEM((16,), jnp.int32),
            pltpu.SMEM((8,), jnp.int32),
            pltpu.SemaphoreType.DMA,
        ],
        compiler_params=_sc_params,
    )
    def k(src_hbm, dst_hbm, srcb_hbm, dstb_hbm, cnts_hbm, starts_hbm,
          src_v, dst_v, tot_v, st_v, stg_s, stg_d, mv, sm, sem):
        t = _wid()
        nb = jnp.int32(_NB)
        tot_v[pl.ds(0, 16)] = jnp.zeros((16,), jnp.int32)
        tot_v[pl.ds(16, 16)] = jnp.zeros((16,), jnp.int32)
        tot_v[pl.ds(32, 16)] = jnp.zeros((16,), jnp.int32)
        ones = jnp.ones((16,), jnp.int32)

        @pl.loop(0, n_ch)
        def _(c):
            pltpu.sync_copy(dst_hbm.at[pl.ds(c * CH2, CH2)], dst_v)

            @pl.loop(0, CH2, step=16)
            def _(g):
                b16 = dst_v[pl.ds(g, 16)] // nb
                plsc.addupdate_scatter(tot_v, [b16], ones)

        t0 = tot_v[pl.ds(0, 16)]
        t1 = tot_v[pl.ds(16, 16)]
        p0 = ((t0 + 255) // 256) * 256
        p1 = ((t1 + 255) // 256) * 256
        c0 = plsc.cumsum(p0)
        c1 = plsc.cumsum(p1) + c0[15]
        st_v[pl.ds(0, 16)] = c0 - p0
        st_v[pl.ds(16, 16)] = c1 - p1
        tsp = jnp.full((16,), t, jnp.int32)
        my_start = plsc.load_gather(st_v, [tsp])[0]
        mv[pl.ds(0, 16)] = plsc.load_gather(tot_v, [tsp])
        pltpu.sync_copy(mv, cnts_hbm.at[pl.ds(t * 16, 16)])
        mv[pl.ds(0, 16)] = plsc.load_gather(st_v, [tsp])
        pltpu.sync_copy(mv, starts_hbm.at[pl.ds(t * 16, 16)])

        sm[0] = 0
        sm[1] = my_start

        @pl.loop(0, n_ch)
        def _(c):
            pltpu.sync_copy(src_hbm.at[pl.ds(c * CH2, CH2)], src_v)
            pltpu.sync_copy(dst_hbm.at[pl.ds(c * CH2, CH2)], dst_v)

            @pl.loop(0, CH2, step=16)
            def _(g):
                d16 = dst_v[pl.ds(g, 16)]
                s16 = src_v[pl.ds(g, 16)]
                m = (d16 // nb) == t
                cs = plsc.cumsum(m.astype(jnp.int32))
                fill = sm[0]
                pos = fill + cs - 1
                plsc.store_scatter(stg_s, [pos], s16, mask=m)
                plsc.store_scatter(stg_d, [pos], d16, mask=m)
                sm[0] = fill + cs[15]

                @pl.when(sm[0] >= 256)
                def _():
                    w = pl.multiple_of(sm[1], 256)
                    pltpu.sync_copy(stg_s.at[pl.ds(0, 256)],
                                    srcb_hbm.at[pl.ds(w, 256)])
                    pltpu.sync_copy(stg_d.at[pl.ds(0, 256)],
                                    dstb_hbm.at[pl.ds(w, 256)])
                    stg_s[pl.ds(0, 16)] = stg_s[pl.ds(256, 16)]
                    stg_d[pl.ds(0, 16)] = stg_d[pl.ds(256, 16)]
                    sm[0] = sm[0] - 256
                    sm[1] = w + 256

        @pl.when(sm[0] > 0)
        def _():
            fill = sm[0]
            sent_d = jnp.full((16,), (t + 1) * _NB, jnp.int32)
            zeros = jnp.zeros((16,), jnp.int32)

            @pl.loop(0, 256, step=16)
            def _(j):
                iv = _iota16() + j
                mm = iv >= fill
                plsc.store_scatter(stg_s, [iv], zeros, mask=mm)
                plsc.store_scatter(stg_d, [iv], sent_d, mask=mm)
            w = pl.multiple_of(sm[1], 256)
            pltpu.sync_copy(stg_s.at[pl.ds(0, 256)],
                            srcb_hbm.at[pl.ds(w, 256)])
            pltpu.sync_copy(stg_d.at[pl.ds(0, 256)],
                            dstb_hbm.at[pl.ds(w, 256)])

    return k(src, dst)


def _sc_gat_layer(srcb, dstb, cnts, starts, z0, z1, z2, z3, s_tab, t_tab, EP):
    @functools.partial(
        pl.kernel, mesh=_mesh(),
        out_type=(jax.ShapeDtypeStruct((_NP * 20,), jnp.float32),
                  jax.ShapeDtypeStruct((_NP * 20,), jnp.float32),
                  jax.ShapeDtypeStruct((_NP * 20,), jnp.float32),
                  jax.ShapeDtypeStruct((_NP * 20,), jnp.float32),
                  jax.ShapeDtypeStruct((EP * 4,), jnp.float32)),
        scratch_types=[
            pltpu.VMEM((_NB * 4 + 16,), jnp.float32),
            pltpu.VMEM((_NB * 4 + 16,), jnp.float32),
            pltpu.VMEM((_NB * 20 + 32,), jnp.float32),
            pltpu.VMEM((_CH,), jnp.int32),
            pltpu.VMEM((_CH,), jnp.int32),
            pltpu.VMEM((_CH, 16), jnp.float32),
            pltpu.VMEM((_CH, 32), jnp.float32),
            pltpu.VMEM((_CH * 4,), jnp.float32),
            pltpu.VMEM((16,), jnp.int32),
            pltpu.SemaphoreType.DMA,
        ],
        compiler_params=_sc_params,
    )
    def k(srcb_hbm, dstb_hbm, cnts_hbm, starts_hbm,
          z0_hbm, z1_hbm, z2_hbm, z3_hbm, s_hbm, t_hbm,
          o0_hbm, o1_hbm, o2_hbm, o3_hbm, ee_hbm,
          t_loc, den, acc, src_v, dst_v, sg, zg, eebuf, mv, sem):
        t = _wid()
        base_n = t * _NB

        pltpu.sync_copy(cnts_hbm.at[pl.ds(t * 16, 16)], mv)
        cnt = mv[pl.ds(0, 16)][0]
        pltpu.sync_copy(starts_hbm.at[pl.ds(t * 16, 16)], mv)
        ep0 = mv[pl.ds(0, 16)][0]
        n_ch = (cnt + _CH - 1) // _CH

        pltpu.sync_copy(t_hbm.at[pl.ds(base_n * 4, _NB * 4)],
                        t_loc.at[pl.ds(0, _NB * 4)])
        _zero_ref(den, _NB * 4 + 16)

        iota = _iota16()

        @pl.loop(0, n_ch)
        def _(c):
            off = pl.multiple_of(ep0 + c * _CH, 256)
            pltpu.sync_copy(srcb_hbm.at[pl.ds(off, _CH)], src_v)
            pltpu.sync_copy(dstb_hbm.at[pl.ds(off, _CH)], dst_v)
            pltpu.async_copy(s_hbm.at[src_v], sg, sem).wait()

            @pl.loop(0, _CH, step=16)
            def _(g):
                rows = iota + g
                dl = dst_v[pl.ds(g, 16)] - base_n
                dl4 = dl * 4
                e4 = rows * 4
                for h in range(4):
                    hsp = jnp.full((16,), h, jnp.int32)
                    s_h = plsc.load_gather(sg, [rows, hsp])
                    t_h = plsc.load_gather(t_loc, [dl4 + h])
                    e = s_h + t_h
                    e = jnp.where(e > 0, e, LRELU_SLOPE * e)
                    ee = jnp.exp(e)
                    plsc.store_scatter(eebuf, [e4 + h], ee)
                    plsc.addupdate_scatter(den, [dl4 + h], ee)
            pltpu.sync_copy(eebuf,
                            ee_hbm.at[pl.ds(pl.multiple_of(off * 4, 1024),
                                            _CH * 4)])

        for h in range(4):
            zh = (z0_hbm, z1_hbm, z2_hbm, z3_hbm)[h]
            oh = (o0_hbm, o1_hbm, o2_hbm, o3_hbm)[h]
            _zero_ref(acc, _NB * 20 + 32)

            @pl.loop(0, n_ch)
            def _(c):
                off = pl.multiple_of(ep0 + c * _CH, 256)
                pltpu.sync_copy(srcb_hbm.at[pl.ds(off, _CH)], src_v)
                pltpu.sync_copy(dstb_hbm.at[pl.ds(off, _CH)], dst_v)
                pltpu.sync_copy(ee_hbm.at[pl.ds(pl.multiple_of(off * 4, 1024),
                                                _CH * 4)], eebuf)
                pltpu.async_copy(zh.at[src_v], zg, sem).wait()

                @pl.loop(0, _CH, step=16)
                def _(g):
                    rows = iota + g
                    dl = dst_v[pl.ds(g, 16)] - base_n
                    ee = plsc.load_gather(eebuf, [rows * 4 + h])
                    dn = plsc.load_gather(den, [dl * 4 + h])
                    att = ee / (dn + 1e-9)
                    dl20 = dl * 20
                    for f in range(20):
                        fsp = jnp.full((16,), f, jnp.int32)
                        col = plsc.load_gather(zg, [rows, fsp])
                        plsc.addupdate_scatter(acc, [dl20 + f], col * att)

            pltpu.sync_copy(acc.at[pl.ds(0, _NB * 20)],
                            oh.at[pl.ds(base_n * 20, _NB * 20)])

    return k(srcb, dstb, cnts, starts, z0, z1, z2, z3, s_tab, t_tab)


def _sc_readout(gh, gids):
    ACC = _G * 20 + 32

    @functools.partial(
        pl.kernel, mesh=_mesh(),
        out_type=jax.ShapeDtypeStruct((_NW * ACC,), jnp.float32),
        scratch_types=[
            pltpu.VMEM((_NB, 20), jnp.float32),
            pltpu.VMEM((_NB,), jnp.int32),
            pltpu.VMEM((ACC,), jnp.float32),
            pltpu.SemaphoreType.DMA,
        ],
        compiler_params=_sc_params,
    )
    def k(gh_hbm, gid_hbm, out_hbm, gh_v, gid_v, acc, sem):
        t = _wid()
        base_n = t * _NB
        pltpu.sync_copy(gh_hbm.at[pl.ds(base_n, _NB)], gh_v)
        pltpu.sync_copy(gid_hbm.at[pl.ds(base_n, _NB)], gid_v)
        _zero_ref(acc, ACC)
        iota = _iota16()

        @pl.loop(0, _NB, step=16)
        def _(g):
            rows = iota + g
            gid20 = gid_v[pl.ds(g, 16)] * 20
            for f in range(20):
                fsp = jnp.full((16,), f, jnp.int32)
                col = plsc.load_gather(gh_v, [rows, fsp])
                plsc.addupdate_scatter(acc, [gid20 + f], col)
        pltpu.sync_copy(acc, out_hbm.at[pl.ds(t * ACC, ACC)])

    return k(gh, gids)


def _tc_mlp_proj(x, w1, b1, w2, b2, wg, asr, ads, bn=2048):
    npad, dp = x.shape
    h1 = w1.shape[1]

    def body(x_ref, w1_ref, b1_ref, w2_ref, b2_ref, wg_ref, as_ref, ad_ref,
             z0_ref, z1_ref, z2_ref, z3_ref, s_ref, t_ref):
        hh = jnp.dot(x_ref[...], w1_ref[...],
                     preferred_element_type=jnp.float32) + b1_ref[...]
        hh = jnp.maximum(hh, 0.0)
        h = jnp.dot(hh, w2_ref[...],
                    preferred_element_type=jnp.float32) + b2_ref[...]
        _proj_common(h, wg_ref, as_ref, ad_ref,
                     (z0_ref, z1_ref, z2_ref, z3_ref), s_ref, t_ref)

    return pl.pallas_call(
        body,
        grid=(npad // bn,),
        in_specs=[
            pl.BlockSpec((bn, dp), lambda i: (i, 0)),
            pl.BlockSpec((dp, h1), lambda i: (0, 0)),
            pl.BlockSpec((1, h1), lambda i: (0, 0)),
            pl.BlockSpec((h1, 20), lambda i: (0, 0)),
            pl.BlockSpec((1, 20), lambda i: (0, 0)),
            pl.BlockSpec((80, 32), lambda i: (0, 0)),
            pl.BlockSpec((8, 32), lambda i: (0, 0)),
            pl.BlockSpec((8, 32), lambda i: (0, 0)),
        ],
        out_specs=[
            pl.BlockSpec((bn, 32), lambda i: (i, 0)),
            pl.BlockSpec((bn, 32), lambda i: (i, 0)),
            pl.BlockSpec((bn, 32), lambda i: (i, 0)),
            pl.BlockSpec((bn, 32), lambda i: (i, 0)),
            pl.BlockSpec((bn, 16), lambda i: (i, 0)),
            pl.BlockSpec((bn, 4), lambda i: (i, 0)),
        ],
        out_shape=[
            jax.ShapeDtypeStruct((npad, 32), jnp.float32),
            jax.ShapeDtypeStruct((npad, 32), jnp.float32),
            jax.ShapeDtypeStruct((npad, 32), jnp.float32),
            jax.ShapeDtypeStruct((npad, 32), jnp.float32),
            jax.ShapeDtypeStruct((npad, 16), jnp.float32),
            jax.ShapeDtypeStruct((npad, 4), jnp.float32),
        ],
    )(x, w1, b1, w2, b2, wg, asr, ads)


def _proj_common(h, wg_ref, as_ref, ad_ref, z_refs, s_ref, t_ref):
    bn = h.shape[0]
    wg = wg_ref[...]
    av = as_ref[...]
    bv = ad_ref[...]
    ss, tt = [], []
    for hd in range(4):
        w = wg[hd * 20:(hd + 1) * 20, :]
        z = jnp.dot(h, w, preferred_element_type=jnp.float32)
        z_refs[hd][...] = z
        ss.append(jnp.sum(z * av[hd, :][None, :], axis=1, keepdims=True))
        tt.append(jnp.sum(z * bv[hd, :][None, :], axis=1, keepdims=True))
    zeros = jnp.zeros((bn, 12), jnp.float32)
    s_ref[...] = jnp.concatenate(ss + [zeros], axis=1)
    t_ref[...] = jnp.concatenate(tt, axis=1)


def _tc_mean_proj(o0, o1, o2, o3, wg, asr, ads, bn=2048):
    npad = o0.shape[0]

    def body(a_ref, b_ref, c_ref, d_ref, wg_ref, as_ref, ad_ref,
             z0_ref, z1_ref, z2_ref, z3_ref, s_ref, t_ref):
        m = 0.25 * (a_ref[...] + b_ref[...] + c_ref[...] + d_ref[...])
        h = jnp.where(m > 0, m, jnp.exp(jnp.minimum(m, 0.0)) - 1.0)
        _proj_common(h, wg_ref, as_ref, ad_ref,
                     (z0_ref, z1_ref, z2_ref, z3_ref), s_ref, t_ref)

    return pl.pallas_call(
        body,
        grid=(npad // bn,),
        in_specs=[pl.BlockSpec((bn, 20), lambda i: (i, 0))] * 4 + [
            pl.BlockSpec((80, 32), lambda i: (0, 0)),
            pl.BlockSpec((8, 32), lambda i: (0, 0)),
            pl.BlockSpec((8, 32), lambda i: (0, 0)),
        ],
        out_specs=[
            pl.BlockSpec((bn, 32), lambda i: (i, 0)),
            pl.BlockSpec((bn, 32), lambda i: (i, 0)),
            pl.BlockSpec((bn, 32), lambda i: (i, 0)),
            pl.BlockSpec((bn, 32), lambda i: (i, 0)),
            pl.BlockSpec((bn, 16), lambda i: (i, 0)),
            pl.BlockSpec((bn, 4), lambda i: (i, 0)),
        ],
        out_shape=[
            jax.ShapeDtypeStruct((npad, 32), jnp.float32),
            jax.ShapeDtypeStruct((npad, 32), jnp.float32),
            jax.ShapeDtypeStruct((npad, 32), jnp.float32),
            jax.ShapeDtypeStruct((npad, 32), jnp.float32),
            jax.ShapeDtypeStruct((npad, 16), jnp.float32),
            jax.ShapeDtypeStruct((npad, 4), jnp.float32),
        ],
    )(o0, o1, o2, o3, wg, asr, ads)


def _tc_mean_gate(o0, o1, o2, o3, w_gate, b_gate, bn=2048):
    npad = o0.shape[0]

    def body(a_ref, b_ref, c_ref, d_ref, wg_ref, bg_ref, gh_ref):
        m = 0.25 * (a_ref[...] + b_ref[...] + c_ref[...] + d_ref[...])
        h = jnp.where(m > 0, m, jnp.exp(jnp.minimum(m, 0.0)) - 1.0)
        gl = jnp.sum(h * wg_ref[...], axis=1, keepdims=True) + bg_ref[...]
        gate = 1.0 / (1.0 + jnp.exp(-gl))
        gh_ref[...] = gate * h

    return pl.pallas_call(
        body,
        grid=(npad // bn,),
        in_specs=[pl.BlockSpec((bn, 20), lambda i: (i, 0))] * 4 + [
            pl.BlockSpec((1, 20), lambda i: (0, 0)),
            pl.BlockSpec((1, 1), lambda i: (0, 0)),
        ],
        out_specs=pl.BlockSpec((bn, 20), lambda i: (i, 0)),
        out_shape=jax.ShapeDtypeStruct((npad, 20), jnp.float32),
    )(o0, o1, o2, o3, w_gate, b_gate)


def _tc_reduce_partials(p):
    nw, acc = p.shape

    def body(p_ref, o_ref):
        o_ref[...] = jnp.sum(p_ref[...], axis=0, keepdims=True)

    return pl.pallas_call(
        body,
        grid=(1,),
        in_specs=[pl.BlockSpec((nw, acc), lambda i: (0, 0))],
        out_specs=pl.BlockSpec((1, acc), lambda i: (0, 0)),
        out_shape=jax.ShapeDtypeStruct((1, acc), jnp.float32),
    )(p)


def kernel(node_ids, edge_index, graph_ids, emb, W1, b1, W2, b2, Wg,
           a_src, a_dst, w_gate, b_gate):
    N = node_ids.shape[0]
    D = emb.shape[1]
    E = edge_index.shape[1]
    EP = E + _NW * _CH

    DP = ((D + 15) // 16) * 16
    embp = jnp.pad(emb, ((0, 0), (0, DP - D)))
    idsp = jnp.pad(node_ids, (0, _NP - N)).astype(jnp.int32)
    w1p = jnp.pad(W1, ((0, DP - D), (0, 0)))
    src = edge_index[0].astype(jnp.int32)
    dst = edge_index[1].astype(jnp.int32)
    wg_l = [jnp.pad(Wg[l].reshape(80, 20), ((0, 0), (0, 12))) for l in range(2)]
    as_l = [jnp.pad(a_src[l], ((0, 4), (0, 12))) for l in range(2)]
    ad_l = [jnp.pad(a_dst[l], ((0, 4), (0, 12))) for l in range(2)]
    gidp = jnp.pad(graph_ids, (0, _NP - N), constant_values=_G).astype(jnp.int32)

    x = _sc_gather_rows(embp, idsp, n_chunks=16, chunk=_NP // _NW // 16)
    srcb, dstb, cnts, starts = _sc_bucket(src, dst)

    z0, z1, z2, z3, s_tab, t_tab = _tc_mlp_proj(
        x, w1p, b1[None, :], W2, b2[None, :], wg_l[0], as_l[0], ad_l[0])
    t_flat = t_tab.reshape(-1)

    o0, o1, o2, o3, _ = _sc_gat_layer(
        srcb, dstb, cnts, starts, z0, z1, z2, z3, s_tab, t_flat, EP)
    o = [r.reshape(_NP, 20) for r in (o0, o1, o2, o3)]

    z0, z1, z2, z3, s_tab, t_tab = _tc_mean_proj(
        o[0], o[1], o[2], o[3], wg_l[1], as_l[1], ad_l[1])

    o0, o1, o2, o3, _ = _sc_gat_layer(
        srcb, dstb, cnts, starts, z0, z1, z2, z3, s_tab,
        t_tab.reshape(-1), EP)
    o = [r.reshape(_NP, 20) for r in (o0, o1, o2, o3)]

    gh = _tc_mean_gate(o[0], o[1], o[2], o[3], w_gate.reshape(1, 20),
                       b_gate.reshape(1, 1))
    partials = _sc_readout(gh, gidp).reshape(_NW, _G * 20 + 32)
    out = _tc_reduce_partials(partials)[0, :_G * 20].reshape(_G, 20)
    return out


# speedup vs baseline: 15.4104x; 1.0602x over previous
import functools

import jax
import jax.numpy as jnp
from jax import lax
from jax.experimental import pallas as pl
from jax.experimental.pallas import tpu as pltpu
from jax.experimental.pallas import tpu_sc as plsc

LRELU_SLOPE = 0.2

_NW = 32
_NB = 3200
_NP = _NW * _NB
_CH = 256
_G = 512

_sc_params = pltpu.CompilerParams(needs_layout_passes=False,
                                 use_tc_tiling_on_sc=False)


def _mesh():
    return plsc.VectorSubcoreMesh(core_axis_name="c", subcore_axis_name="s")


def _wid():
    return lax.axis_index("s") * 2 + lax.axis_index("c")


def _iota16():
    return lax.iota(jnp.int32, 16)


def _zero_ref(ref, n):
    @pl.loop(0, n, step=16)
    def _(i):
        ref[pl.ds(i, 16)] = jnp.zeros((16,), jnp.float32)


def _sc_gather_rows(table, idx, n_chunks, chunk):
    npad, dp = idx.shape[0], table.shape[1]
    per_w = n_chunks * chunk
    assert npad == _NW * per_w

    @functools.partial(
        pl.kernel, mesh=_mesh(),
        out_type=jax.ShapeDtypeStruct((npad, dp), jnp.float32),
        scratch_types=[
            pltpu.VMEM((chunk,), jnp.int32),
            pltpu.VMEM((chunk, dp), jnp.float32),
            pltpu.SemaphoreType.DMA,
        ],
        compiler_params=_sc_params,
    )
    def k(table_hbm, idx_hbm, out_hbm, idx_v, rows_v, sem):
        wid = _wid()

        @pl.loop(0, n_chunks)
        def _(c):
            base = wid * per_w + c * chunk
            pltpu.sync_copy(idx_hbm.at[pl.ds(base, chunk)], idx_v)
            pltpu.async_copy(table_hbm.at[idx_v], rows_v, sem).wait()
            pltpu.sync_copy(rows_v, out_hbm.at[pl.ds(base, chunk)])

    return k(table, idx)


def _sc_bucket(src, dst):
    E = src.shape[0]
    CH2 = 8000
    n_ch = E // CH2
    assert n_ch * CH2 == E
    EP = E + _NW * _CH

    @functools.partial(
        pl.kernel, mesh=_mesh(),
        out_type=(jax.ShapeDtypeStruct((EP,), jnp.int32),
                  jax.ShapeDtypeStruct((EP,), jnp.int32),
                  jax.ShapeDtypeStruct((_NW * 16,), jnp.int32),
                  jax.ShapeDtypeStruct((_NW * 16,), jnp.int32)),
        scratch_types=[
            pltpu.VMEM((CH2,), jnp.int32),
            pltpu.VMEM((CH2,), jnp.int32),
            pltpu.VMEM((48,), jnp.int32),
            pltpu.VMEM((48,), jnp.int32),
            pltpu.VMEM((272,), jnp.int32),
            pltpu.VMEM((272,), jnp.int32),
            pltpu.VMEM((16,), jnp.int32),
            pltpu.SMEM((8,), jnp.int32),
            pltpu.SemaphoreType.DMA,
            pltpu.SemaphoreType.DMA,
        ],
        compiler_params=_sc_params,
    )
    def k(src_hbm, dst_hbm, srcb_hbm, dstb_hbm, cnts_hbm, starts_hbm,
          src_v, dst_v, tot_v, st_v, stg_s, stg_d, mv, sm, sem, sem2):
        t = _wid()
        nb = jnp.int32(_NB)
        tot_v[pl.ds(0, 16)] = jnp.zeros((16,), jnp.int32)
        tot_v[pl.ds(16, 16)] = jnp.zeros((16,), jnp.int32)
        tot_v[pl.ds(32, 16)] = jnp.zeros((16,), jnp.int32)
        ones = jnp.ones((16,), jnp.int32)

        @pl.loop(0, n_ch)
        def _(c):
            pltpu.sync_copy(dst_hbm.at[pl.ds(c * CH2, CH2)], dst_v)

            @pl.loop(0, CH2, step=16)
            def _(g):
                b16 = dst_v[pl.ds(g, 16)] // nb
                plsc.addupdate_scatter(tot_v, [b16], ones)

        t0 = tot_v[pl.ds(0, 16)]
        t1 = tot_v[pl.ds(16, 16)]
        p0 = ((t0 + 255) // 256) * 256
        p1 = ((t1 + 255) // 256) * 256
        c0 = plsc.cumsum(p0)
        c1 = plsc.cumsum(p1) + c0[15]
        st_v[pl.ds(0, 16)] = c0 - p0
        st_v[pl.ds(16, 16)] = c1 - p1
        tsp = jnp.full((16,), t, jnp.int32)
        my_start = plsc.load_gather(st_v, [tsp])[0]
        mv[pl.ds(0, 16)] = plsc.load_gather(tot_v, [tsp])
        pltpu.sync_copy(mv, cnts_hbm.at[pl.ds(t * 16, 16)])
        mv[pl.ds(0, 16)] = plsc.load_gather(st_v, [tsp])
        pltpu.sync_copy(mv, starts_hbm.at[pl.ds(t * 16, 16)])

        sm[0] = 0
        sm[1] = my_start

        @pl.loop(0, n_ch)
        def _(c):
            h1 = pltpu.async_copy(src_hbm.at[pl.ds(c * CH2, CH2)], src_v, sem)
            h2 = pltpu.async_copy(dst_hbm.at[pl.ds(c * CH2, CH2)], dst_v, sem2)
            h1.wait()
            h2.wait()

            @pl.loop(0, CH2, step=16)
            def _(g):
                d16 = dst_v[pl.ds(g, 16)]
                m = (d16 // nb) == t

                @pl.when(jnp.any(m))
                def _():
                    s16 = src_v[pl.ds(g, 16)]
                    cs = plsc.cumsum(m.astype(jnp.int32))
                    fill = sm[0]
                    pos = fill + cs - 1
                    plsc.store_scatter(stg_s, [pos], s16, mask=m)
                    plsc.store_scatter(stg_d, [pos], d16, mask=m)
                    sm[0] = fill + cs[15]

                @pl.when(sm[0] >= 256)
                def _():
                    w = pl.multiple_of(sm[1], 256)
                    pltpu.sync_copy(stg_s.at[pl.ds(0, 256)],
                                    srcb_hbm.at[pl.ds(w, 256)])
                    pltpu.sync_copy(stg_d.at[pl.ds(0, 256)],
                                    dstb_hbm.at[pl.ds(w, 256)])
                    stg_s[pl.ds(0, 16)] = stg_s[pl.ds(256, 16)]
                    stg_d[pl.ds(0, 16)] = stg_d[pl.ds(256, 16)]
                    sm[0] = sm[0] - 256
                    sm[1] = w + 256

        @pl.when(sm[0] > 0)
        def _():
            fill = sm[0]
            sent_d = jnp.full((16,), (t + 1) * _NB, jnp.int32)
            zeros = jnp.zeros((16,), jnp.int32)

            @pl.loop(0, 256, step=16)
            def _(j):
                iv = _iota16() + j
                mm = iv >= fill
                plsc.store_scatter(stg_s, [iv], zeros, mask=mm)
                plsc.store_scatter(stg_d, [iv], sent_d, mask=mm)
            w = pl.multiple_of(sm[1], 256)
            pltpu.sync_copy(stg_s.at[pl.ds(0, 256)],
                            srcb_hbm.at[pl.ds(w, 256)])
            pltpu.sync_copy(stg_d.at[pl.ds(0, 256)],
                            dstb_hbm.at[pl.ds(w, 256)])

    return k(src, dst)


def _sc_gat_layer(srcb, dstb, cnts, starts, z0, z1, z2, z3, s_tab, t_tab, EP):
    @functools.partial(
        pl.kernel, mesh=_mesh(),
        out_type=(jax.ShapeDtypeStruct((_NP * 20,), jnp.float32),
                  jax.ShapeDtypeStruct((_NP * 20,), jnp.float32),
                  jax.ShapeDtypeStruct((_NP * 20,), jnp.float32),
                  jax.ShapeDtypeStruct((_NP * 20,), jnp.float32),
                  jax.ShapeDtypeStruct((EP * 4,), jnp.float32)),
        scratch_types=[
            pltpu.VMEM((_NB * 4 + 16,), jnp.float32),
            pltpu.VMEM((_NB * 4 + 16,), jnp.float32),
            pltpu.VMEM((_NB * 20 + 32,), jnp.float32),
            pltpu.VMEM((_CH,), jnp.int32),
            pltpu.VMEM((_CH,), jnp.int32),
            pltpu.VMEM((_CH, 16), jnp.float32),
            pltpu.VMEM((_CH, 32), jnp.float32),
            pltpu.VMEM((_CH * 4,), jnp.float32),
            pltpu.VMEM((16,), jnp.int32),
            pltpu.SemaphoreType.DMA,
            pltpu.SemaphoreType.DMA,
            pltpu.SemaphoreType.DMA,
            pltpu.SemaphoreType.DMA,
        ],
        compiler_params=_sc_params,
    )
    def k(srcb_hbm, dstb_hbm, cnts_hbm, starts_hbm,
          z0_hbm, z1_hbm, z2_hbm, z3_hbm, s_hbm, t_hbm,
          o0_hbm, o1_hbm, o2_hbm, o3_hbm, ee_hbm,
          t_loc, den, acc, src_v, dst_v, sg, zg, eebuf, mv,
          sem, sem2, sem3, sem4):
        t = _wid()
        base_n = t * _NB

        pltpu.sync_copy(cnts_hbm.at[pl.ds(t * 16, 16)], mv)
        cnt = mv[pl.ds(0, 16)][0]
        pltpu.sync_copy(starts_hbm.at[pl.ds(t * 16, 16)], mv)
        ep0 = mv[pl.ds(0, 16)][0]
        n_ch = (cnt + _CH - 1) // _CH

        pltpu.sync_copy(t_hbm.at[pl.ds(base_n * 4, _NB * 4)],
                        t_loc.at[pl.ds(0, _NB * 4)])
        _zero_ref(den, _NB * 4 + 16)

        iota = _iota16()

        @pl.loop(0, n_ch)
        def _(c):
            off = pl.multiple_of(ep0 + c * _CH, 256)
            h1 = pltpu.async_copy(srcb_hbm.at[pl.ds(off, _CH)], src_v, sem)
            h2 = pltpu.async_copy(dstb_hbm.at[pl.ds(off, _CH)], dst_v, sem2)
            h1.wait()
            h3 = pltpu.async_copy(s_hbm.at[src_v], sg, sem3)
            h2.wait()
            h3.wait()

            @pl.loop(0, _CH, step=16)
            def _(g):
                rows = iota + g
                dl = dst_v[pl.ds(g, 16)] - base_n
                dl4 = dl * 4
                e4 = rows * 4
                for h in range(4):
                    hsp = jnp.full((16,), h, jnp.int32)
                    s_h = plsc.load_gather(sg, [rows, hsp])
                    t_h = plsc.load_gather(t_loc, [dl4 + h])
                    e = s_h + t_h
                    e = jnp.where(e > 0, e, LRELU_SLOPE * e)
                    ee = jnp.exp(e)
                    plsc.store_scatter(eebuf, [e4 + h], ee)
                    plsc.addupdate_scatter(den, [dl4 + h], ee)
            pltpu.sync_copy(eebuf,
                            ee_hbm.at[pl.ds(pl.multiple_of(off * 4, 1024),
                                            _CH * 4)])

        for h in range(4):
            zh = (z0_hbm, z1_hbm, z2_hbm, z3_hbm)[h]
            oh = (o0_hbm, o1_hbm, o2_hbm, o3_hbm)[h]
            _zero_ref(acc, _NB * 20 + 32)

            @pl.loop(0, n_ch)
            def _(c):
                off = pl.multiple_of(ep0 + c * _CH, 256)
                h1 = pltpu.async_copy(srcb_hbm.at[pl.ds(off, _CH)], src_v, sem)
                h2 = pltpu.async_copy(dstb_hbm.at[pl.ds(off, _CH)], dst_v,
                                      sem2)
                h4 = pltpu.async_copy(
                    ee_hbm.at[pl.ds(pl.multiple_of(off * 4, 1024), _CH * 4)],
                    eebuf, sem4)
                h1.wait()
                h3 = pltpu.async_copy(zh.at[src_v], zg, sem3)
                h2.wait()
                h4.wait()
                h3.wait()

                @pl.loop(0, _CH, step=16)
                def _(g):
                    rows = iota + g
                    dl = dst_v[pl.ds(g, 16)] - base_n
                    ee = plsc.load_gather(eebuf, [rows * 4 + h])
                    dn = plsc.load_gather(den, [dl * 4 + h])
                    att = ee / (dn + 1e-9)
                    dl20 = dl * 20
                    for f in range(20):
                        fsp = jnp.full((16,), f, jnp.int32)
                        col = plsc.load_gather(zg, [rows, fsp])
                        plsc.addupdate_scatter(acc, [dl20 + f], col * att)

            pltpu.sync_copy(acc.at[pl.ds(0, _NB * 20)],
                            oh.at[pl.ds(base_n * 20, _NB * 20)])

    return k(srcb, dstb, cnts, starts, z0, z1, z2, z3, s_tab, t_tab)


def _sc_readout(gh, gids):
    ACC = _G * 20 + 32

    @functools.partial(
        pl.kernel, mesh=_mesh(),
        out_type=jax.ShapeDtypeStruct((_NW * ACC,), jnp.float32),
        scratch_types=[
            pltpu.VMEM((_NB, 20), jnp.float32),
            pltpu.VMEM((_NB,), jnp.int32),
            pltpu.VMEM((ACC,), jnp.float32),
            pltpu.SemaphoreType.DMA,
        ],
        compiler_params=_sc_params,
    )
    def k(gh_hbm, gid_hbm, out_hbm, gh_v, gid_v, acc, sem):
        t = _wid()
        base_n = t * _NB
        pltpu.sync_copy(gh_hbm.at[pl.ds(base_n, _NB)], gh_v)
        pltpu.sync_copy(gid_hbm.at[pl.ds(base_n, _NB)], gid_v)
        _zero_ref(acc, ACC)
        iota = _iota16()

        @pl.loop(0, _NB, step=16)
        def _(g):
            rows = iota + g
            gid20 = gid_v[pl.ds(g, 16)] * 20
            for f in range(20):
                fsp = jnp.full((16,), f, jnp.int32)
                col = plsc.load_gather(gh_v, [rows, fsp])
                plsc.addupdate_scatter(acc, [gid20 + f], col)
        pltpu.sync_copy(acc, out_hbm.at[pl.ds(t * ACC, ACC)])

    return k(gh, gids)


def _tc_mlp_proj(x, w1, b1, w2, b2, wg, asr, ads, bn=2048):
    npad, dp = x.shape
    h1 = w1.shape[1]

    def body(x_ref, w1_ref, b1_ref, w2_ref, b2_ref, wg_ref, as_ref, ad_ref,
             z0_ref, z1_ref, z2_ref, z3_ref, s_ref, t_ref):
        hh = jnp.dot(x_ref[...], w1_ref[...],
                     preferred_element_type=jnp.float32) + b1_ref[...]
        hh = jnp.maximum(hh, 0.0)
        h = jnp.dot(hh, w2_ref[...],
                    preferred_element_type=jnp.float32) + b2_ref[...]
        _proj_common(h, wg_ref, as_ref, ad_ref,
                     (z0_ref, z1_ref, z2_ref, z3_ref), s_ref, t_ref)

    return pl.pallas_call(
        body,
        grid=(npad // bn,),
        in_specs=[
            pl.BlockSpec((bn, dp), lambda i: (i, 0)),
            pl.BlockSpec((dp, h1), lambda i: (0, 0)),
            pl.BlockSpec((1, h1), lambda i: (0, 0)),
            pl.BlockSpec((h1, 20), lambda i: (0, 0)),
            pl.BlockSpec((1, 20), lambda i: (0, 0)),
            pl.BlockSpec((80, 32), lambda i: (0, 0)),
            pl.BlockSpec((8, 32), lambda i: (0, 0)),
            pl.BlockSpec((8, 32), lambda i: (0, 0)),
        ],
        out_specs=[
            pl.BlockSpec((bn, 32), lambda i: (i, 0)),
            pl.BlockSpec((bn, 32), lambda i: (i, 0)),
            pl.BlockSpec((bn, 32), lambda i: (i, 0)),
            pl.BlockSpec((bn, 32), lambda i: (i, 0)),
            pl.BlockSpec((bn, 16), lambda i: (i, 0)),
            pl.BlockSpec((bn, 4), lambda i: (i, 0)),
        ],
        out_shape=[
            jax.ShapeDtypeStruct((npad, 32), jnp.float32),
            jax.ShapeDtypeStruct((npad, 32), jnp.float32),
            jax.ShapeDtypeStruct((npad, 32), jnp.float32),
            jax.ShapeDtypeStruct((npad, 32), jnp.float32),
            jax.ShapeDtypeStruct((npad, 16), jnp.float32),
            jax.ShapeDtypeStruct((npad, 4), jnp.float32),
        ],
    )(x, w1, b1, w2, b2, wg, asr, ads)


def _proj_common(h, wg_ref, as_ref, ad_ref, z_refs, s_ref, t_ref):
    bn = h.shape[0]
    wg = wg_ref[...]
    av = as_ref[...]
    bv = ad_ref[...]
    ss, tt = [], []
    for hd in range(4):
        w = wg[hd * 20:(hd + 1) * 20, :]
        z = jnp.dot(h, w, preferred_element_type=jnp.float32)
        z_refs[hd][...] = z
        ss.append(jnp.sum(z * av[hd, :][None, :], axis=1, keepdims=True))
        tt.append(jnp.sum(z * bv[hd, :][None, :], axis=1, keepdims=True))
    zeros = jnp.zeros((bn, 12), jnp.float32)
    s_ref[...] = jnp.concatenate(ss + [zeros], axis=1)
    t_ref[...] = jnp.concatenate(tt, axis=1)


def _tc_mean_proj(o0, o1, o2, o3, wg, asr, ads, bn=2048):
    npad = o0.shape[0]

    def body(a_ref, b_ref, c_ref, d_ref, wg_ref, as_ref, ad_ref,
             z0_ref, z1_ref, z2_ref, z3_ref, s_ref, t_ref):
        m = 0.25 * (a_ref[...] + b_ref[...] + c_ref[...] + d_ref[...])
        h = jnp.where(m > 0, m, jnp.exp(jnp.minimum(m, 0.0)) - 1.0)
        _proj_common(h, wg_ref, as_ref, ad_ref,
                     (z0_ref, z1_ref, z2_ref, z3_ref), s_ref, t_ref)

    return pl.pallas_call(
        body,
        grid=(npad // bn,),
        in_specs=[pl.BlockSpec((bn, 20), lambda i: (i, 0))] * 4 + [
            pl.BlockSpec((80, 32), lambda i: (0, 0)),
            pl.BlockSpec((8, 32), lambda i: (0, 0)),
            pl.BlockSpec((8, 32), lambda i: (0, 0)),
        ],
        out_specs=[
            pl.BlockSpec((bn, 32), lambda i: (i, 0)),
            pl.BlockSpec((bn, 32), lambda i: (i, 0)),
            pl.BlockSpec((bn, 32), lambda i: (i, 0)),
            pl.BlockSpec((bn, 32), lambda i: (i, 0)),
            pl.BlockSpec((bn, 16), lambda i: (i, 0)),
            pl.BlockSpec((bn, 4), lambda i: (i, 0)),
        ],
        out_shape=[
            jax.ShapeDtypeStruct((npad, 32), jnp.float32),
            jax.ShapeDtypeStruct((npad, 32), jnp.float32),
            jax.ShapeDtypeStruct((npad, 32), jnp.float32),
            jax.ShapeDtypeStruct((npad, 32), jnp.float32),
            jax.ShapeDtypeStruct((npad, 16), jnp.float32),
            jax.ShapeDtypeStruct((npad, 4), jnp.float32),
        ],
    )(o0, o1, o2, o3, wg, asr, ads)


def _tc_mean_gate(o0, o1, o2, o3, w_gate, b_gate, bn=2048):
    npad = o0.shape[0]

    def body(a_ref, b_ref, c_ref, d_ref, wg_ref, bg_ref, gh_ref):
        m = 0.25 * (a_ref[...] + b_ref[...] + c_ref[...] + d_ref[...])
        h = jnp.where(m > 0, m, jnp.exp(jnp.minimum(m, 0.0)) - 1.0)
        gl = jnp.sum(h * wg_ref[...], axis=1, keepdims=True) + bg_ref[...]
        gate = 1.0 / (1.0 + jnp.exp(-gl))
        gh_ref[...] = gate * h

    return pl.pallas_call(
        body,
        grid=(npad // bn,),
        in_specs=[pl.BlockSpec((bn, 20), lambda i: (i, 0))] * 4 + [
            pl.BlockSpec((1, 20), lambda i: (0, 0)),
            pl.BlockSpec((1, 1), lambda i: (0, 0)),
        ],
        out_specs=pl.BlockSpec((bn, 20), lambda i: (i, 0)),
        out_shape=jax.ShapeDtypeStruct((npad, 20), jnp.float32),
    )(o0, o1, o2, o3, w_gate, b_gate)


def _tc_reduce_partials(p):
    nw, acc = p.shape

    def body(p_ref, o_ref):
        o_ref[...] = jnp.sum(p_ref[...], axis=0, keepdims=True)

    return pl.pallas_call(
        body,
        grid=(1,),
        in_specs=[pl.BlockSpec((nw, acc), lambda i: (0, 0))],
        out_specs=pl.BlockSpec((1, acc), lambda i: (0, 0)),
        out_shape=jax.ShapeDtypeStruct((1, acc), jnp.float32),
    )(p)


def kernel(node_ids, edge_index, graph_ids, emb, W1, b1, W2, b2, Wg,
           a_src, a_dst, w_gate, b_gate):
    N = node_ids.shape[0]
    D = emb.shape[1]
    E = edge_index.shape[1]
    EP = E + _NW * _CH

    DP = ((D + 15) // 16) * 16
    embp = jnp.pad(emb, ((0, 0), (0, DP - D)))
    idsp = jnp.pad(node_ids, (0, _NP - N)).astype(jnp.int32)
    w1p = jnp.pad(W1, ((0, DP - D), (0, 0)))
    src = edge_index[0].astype(jnp.int32)
    dst = edge_index[1].astype(jnp.int32)
    wg_l = [jnp.pad(Wg[l].reshape(80, 20), ((0, 0), (0, 12))) for l in range(2)]
    as_l = [jnp.pad(a_src[l], ((0, 4), (0, 12))) for l in range(2)]
    ad_l = [jnp.pad(a_dst[l], ((0, 4), (0, 12))) for l in range(2)]
    gidp = jnp.pad(graph_ids, (0, _NP - N), constant_values=_G).astype(jnp.int32)

    x = _sc_gather_rows(embp, idsp, n_chunks=16, chunk=_NP // _NW // 16)
    srcb, dstb, cnts, starts = _sc_bucket(src, dst)

    z0, z1, z2, z3, s_tab, t_tab = _tc_mlp_proj(
        x, w1p, b1[None, :], W2, b2[None, :], wg_l[0], as_l[0], ad_l[0])
    t_flat = t_tab.reshape(-1)

    o0, o1, o2, o3, _ = _sc_gat_layer(
        srcb, dstb, cnts, starts, z0, z1, z2, z3, s_tab, t_flat, EP)
    o = [r.reshape(_NP, 20) for r in (o0, o1, o2, o3)]

    z0, z1, z2, z3, s_tab, t_tab = _tc_mean_proj(
        o[0], o[1], o[2], o[3], wg_l[1], as_l[1], ad_l[1])

    o0, o1, o2, o3, _ = _sc_gat_layer(
        srcb, dstb, cnts, starts, z0, z1, z2, z3, s_tab,
        t_tab.reshape(-1), EP)
    o = [r.reshape(_NP, 20) for r in (o0, o1, o2, o3)]

    gh = _tc_mean_gate(o[0], o[1], o[2], o[3], w_gate.reshape(1, 20),
                       b_gate.reshape(1, 1))
    partials = _sc_readout(gh, gidp).reshape(_NW, _G * 20 + 32)
    out = _tc_reduce_partials(partials)[0, :_G * 20].reshape(_G, 20)
    return out
